# baseline (speedup 1.0000x reference)
import jax
import jax.numpy as jnp
from jax.experimental import pallas as pl
from jax.experimental.pallas import tpu as pltpu

_SLOPE = 0.01


def _lrelu(a):
    return jnp.where(a > 0.0, a, _SLOPE * a)


def _conv_stack_body(xs_ref, w1_ref, b1_ref, w2_ref, b2_ref, w3_ref, b3_ref,
                     o_ref):
    tb = xs_ref.shape[2]
    xs = xs_ref[...]

    p1 = jnp.concatenate(
        [xs[dy:dy + 20, dx:dx + 20] for dy in range(2) for dx in range(2)],
        axis=-1)
    a1 = jnp.dot(p1.reshape(400 * tb, 192), w1_ref[...],
                 preferred_element_type=jnp.float32) + b1_ref[...]
    h1 = _lrelu(a1).astype(jnp.bfloat16)
    h1 = h1.reshape(10, 2, 10, 2, tb, 32)

    p2 = jnp.concatenate(
        [h1[dy:dy + 9, fy, dx:dx + 9, fx]
         for dy in range(2) for dx in range(2)
         for fy in range(2) for fx in range(2)],
        axis=-1)
    a2 = jnp.dot(p2.reshape(81 * tb, 512), w2_ref[...],
                 preferred_element_type=jnp.float32) + b2_ref[...]
    h2 = _lrelu(a2).astype(jnp.bfloat16).reshape(9, 9, tb, 64)

    p3 = jnp.concatenate(
        [h2[ky:ky + 7, kx:kx + 7] for ky in range(3) for kx in range(3)],
        axis=-1)
    a3 = jnp.dot(p3.reshape(49 * tb, 576), w3_ref[...],
                 preferred_element_type=jnp.float32) + b3_ref[...]
    o_ref[...] = _lrelu(a3).astype(jnp.bfloat16).reshape(49, tb, 64)


def _head_body(x_ref, w1_ref, b1_ref, w2_ref, b2_ref, o_ref):
    a = jnp.dot(x_ref[...], w1_ref[...],
                preferred_element_type=jnp.float32) + b1_ref[...]
    o_ref[...] = jnp.dot(_lrelu(a), w2_ref[...],
                         preferred_element_type=jnp.float32) + b2_ref[...]


def kernel(x_nchw, cw1, cb1, cw2, cb2, cw3, cb3, hw1, hb1, hw2, hb2):
    B = x_nchw.shape[0]
    TB = 64 if B % 64 == 0 else 32
    bf16 = jnp.bfloat16

    xs = (x_nchw.astype(bf16).reshape(B, 3, 21, 4, 21, 4)
          .transpose(2, 4, 0, 3, 1, 5)
          .reshape(21, 21, B, 48))
    w1p = (cw1.reshape(2, 4, 2, 4, 3, 32)
           .transpose(0, 2, 1, 4, 3, 5).reshape(192, 32).astype(bf16))
    w2p = (cw2.reshape(2, 2, 2, 2, 32, 64)
           .transpose(0, 2, 1, 3, 4, 5).reshape(512, 64).astype(bf16))
    w3p = cw3.astype(bf16)
    na = hw2.shape[1] - 2
    w2e = hw2[:, :na] + hw2[:, na:na + 1] - hw2[:, na + 1:na + 2]
    b2e = hb2[:, :na] + hb2[:, na:na + 1] - hb2[:, na + 1:na + 2]

    conv_out = pl.pallas_call(
        _conv_stack_body,
        out_shape=jax.ShapeDtypeStruct((49, B, 64), jnp.bfloat16),
        grid=(B // TB,),
        in_specs=[
            pl.BlockSpec((21, 21, TB, 48), lambda i: (0, 0, i, 0)),
            pl.BlockSpec((192, 32), lambda i: (0, 0)),
            pl.BlockSpec((1, 32), lambda i: (0, 0)),
            pl.BlockSpec((512, 64), lambda i: (0, 0)),
            pl.BlockSpec((1, 64), lambda i: (0, 0)),
            pl.BlockSpec((576, 64), lambda i: (0, 0)),
            pl.BlockSpec((1, 64), lambda i: (0, 0)),
        ],
        out_specs=pl.BlockSpec((49, TB, 64), lambda i: (0, i, 0)),
        compiler_params=pltpu.CompilerParams(
            dimension_semantics=("arbitrary",),
            vmem_limit_bytes=100 * 1024 * 1024),
    )(xs, w1p, cb1, w2p, cb2, w3p, cb3)

    feat = conv_out.transpose(1, 0, 2).reshape(B, 3136)

    TH = 256 if B % 256 == 0 else B
    out = pl.pallas_call(
        _head_body,
        out_shape=jax.ShapeDtypeStruct((B, na), jnp.float32),
        grid=(B // TH,),
        in_specs=[
            pl.BlockSpec((TH, 3136), lambda i: (i, 0)),
            pl.BlockSpec((3136, 1024), lambda i: (0, 0)),
            pl.BlockSpec((1, 1024), lambda i: (0, 0)),
            pl.BlockSpec((1024, na), lambda i: (0, 0)),
            pl.BlockSpec((1, na), lambda i: (0, 0)),
        ],
        out_specs=pl.BlockSpec((TH, na), lambda i: (i, 0)),
        compiler_params=pltpu.CompilerParams(
            dimension_semantics=("arbitrary",),
            vmem_limit_bytes=64 * 1024 * 1024),
    )(feat, hw1.astype(bf16), hb1, w2e, b2e)
    return out


# speedup vs baseline: 1.1138x; 1.1138x over previous
import jax
import jax.numpy as jnp
from jax.experimental import pallas as pl
from jax.experimental.pallas import tpu as pltpu

_SLOPE = 0.01


def _lrelu(a):
    return jnp.where(a > 0.0, a, _SLOPE * a)


def _conv_stack_body(xs_ref, w1_ref, b1_ref, w2_ref, b2_ref, w3_ref, b3_ref,
                     o_ref):
    tb = xs_ref.shape[2]
    xs = xs_ref[...]

    p1 = jnp.concatenate(
        [xs[dy:dy + 20, dx:dx + 20] for dy in range(2) for dx in range(2)],
        axis=-1)
    a1 = jnp.dot(p1.reshape(400 * tb, 192), w1_ref[...],
                 preferred_element_type=jnp.float32) + b1_ref[...]
    h1 = _lrelu(a1).astype(jnp.bfloat16)
    h1 = h1.reshape(10, 2, 10, 2, tb, 32)

    p2 = jnp.concatenate(
        [h1[dy:dy + 9, fy, dx:dx + 9, fx]
         for dy in range(2) for dx in range(2)
         for fy in range(2) for fx in range(2)],
        axis=-1)
    a2 = jnp.dot(p2.reshape(81 * tb, 512), w2_ref[...],
                 preferred_element_type=jnp.float32) + b2_ref[...]
    h2 = _lrelu(a2).astype(jnp.bfloat16).reshape(9, 9, tb, 64)

    p3 = jnp.concatenate(
        [h2[ky:ky + 7, kx:kx + 7] for ky in range(3) for kx in range(3)],
        axis=-1)
    a3 = jnp.dot(p3.reshape(49 * tb, 576), w3_ref[...],
                 preferred_element_type=jnp.float32) + b3_ref[...]
    o_ref[...] = _lrelu(a3).astype(jnp.bfloat16).reshape(49, tb, 64)


def _head_body(x_ref, w1_ref, b1_ref, w2_ref, b2_ref, o_ref):
    a = jnp.dot(x_ref[...], w1_ref[...],
                preferred_element_type=jnp.float32) + b1_ref[...]
    o_ref[...] = jnp.dot(_lrelu(a), w2_ref[...],
                         preferred_element_type=jnp.float32) + b2_ref[...]


def kernel(x_nchw, cw1, cb1, cw2, cb2, cw3, cb3, hw1, hb1, hw2, hb2):
    B = x_nchw.shape[0]
    TB = 64 if B % 64 == 0 else 32
    bf16 = jnp.bfloat16

    xs = (x_nchw.astype(bf16).reshape(B, 3, 21, 4, 21, 4)
          .transpose(2, 4, 0, 3, 5, 1)
          .reshape(21, 21, B, 48))
    w1p = (cw1.reshape(2, 4, 2, 4, 3, 32)
           .transpose(0, 2, 1, 3, 4, 5).reshape(192, 32).astype(bf16))
    w2p = (cw2.reshape(2, 2, 2, 2, 32, 64)
           .transpose(0, 2, 1, 3, 4, 5).reshape(512, 64).astype(bf16))
    w3p = cw3.astype(bf16)
    na = hw2.shape[1] - 2
    w2e = hw2[:, :na] + hw2[:, na:na + 1] - hw2[:, na + 1:na + 2]
    b2e = hb2[:, :na] + hb2[:, na:na + 1] - hb2[:, na + 1:na + 2]

    conv_out = pl.pallas_call(
        _conv_stack_body,
        out_shape=jax.ShapeDtypeStruct((49, B, 64), jnp.bfloat16),
        grid=(B // TB,),
        in_specs=[
            pl.BlockSpec((21, 21, TB, 48), lambda i: (0, 0, i, 0)),
            pl.BlockSpec((192, 32), lambda i: (0, 0)),
            pl.BlockSpec((1, 32), lambda i: (0, 0)),
            pl.BlockSpec((512, 64), lambda i: (0, 0)),
            pl.BlockSpec((1, 64), lambda i: (0, 0)),
            pl.BlockSpec((576, 64), lambda i: (0, 0)),
            pl.BlockSpec((1, 64), lambda i: (0, 0)),
        ],
        out_specs=pl.BlockSpec((49, TB, 64), lambda i: (0, i, 0)),
        compiler_params=pltpu.CompilerParams(
            dimension_semantics=("arbitrary",),
            vmem_limit_bytes=100 * 1024 * 1024),
    )(xs, w1p, cb1, w2p, cb2, w3p, cb3)

    feat = conv_out.transpose(1, 0, 2).reshape(B, 3136)

    TH = 256 if B % 256 == 0 else B
    out = pl.pallas_call(
        _head_body,
        out_shape=jax.ShapeDtypeStruct((B, na), jnp.float32),
        grid=(B // TH,),
        in_specs=[
            pl.BlockSpec((TH, 3136), lambda i: (i, 0)),
            pl.BlockSpec((3136, 1024), lambda i: (0, 0)),
            pl.BlockSpec((1, 1024), lambda i: (0, 0)),
            pl.BlockSpec((1024, na), lambda i: (0, 0)),
            pl.BlockSpec((1, na), lambda i: (0, 0)),
        ],
        out_specs=pl.BlockSpec((TH, na), lambda i: (i, 0)),
        compiler_params=pltpu.CompilerParams(
            dimension_semantics=("arbitrary",),
            vmem_limit_bytes=64 * 1024 * 1024),
    )(feat, hw1.astype(bf16), hb1, w2e, b2e)
    return out
